# baseline (speedup 1.0000x reference)
import functools

import jax
import jax.numpy as jnp
from jax import lax
from jax.experimental import pallas as pl
from jax.experimental.pallas import tpu as pltpu
from jax.experimental.pallas import tpu_sc as plsc

B = 16384
D = 64
NW = 32
BPW = B // NW
C = 128
NCHUNK = BPW // C
L = 16
DJ = D // L

_mesh = plsc.VectorSubcoreMesh(core_axis_name="c", subcore_axis_name="s")


@functools.partial(
    pl.kernel,
    mesh=_mesh,
    compiler_params=pltpu.CompilerParams(
        needs_layout_passes=False, use_tc_tiling_on_sc=False),
    out_type=(
        jax.ShapeDtypeStruct((B,), jnp.float32),
        jax.ShapeDtypeStruct((B,), jnp.float32),
    ),
    scratch_types=[
        pltpu.VMEM((7, NCHUNK, C), jnp.int32),
        pltpu.VMEM((C, D), jnp.float32),
        pltpu.VMEM((C, D), jnp.float32),
        pltpu.VMEM((C, D), jnp.float32),
        pltpu.VMEM((C, D), jnp.float32),
        pltpu.VMEM((C, D), jnp.float32),
        pltpu.VMEM((C, D), jnp.float32),
        pltpu.VMEM((C, D), jnp.float32),
        pltpu.VMEM((C * L,), jnp.float32),
        pltpu.VMEM((C * L,), jnp.float32),
        pltpu.VMEM((BPW,), jnp.float32),
        pltpu.VMEM((BPW,), jnp.float32),
        pltpu.SemaphoreType.DMA,
    ],
)
def _score_kernel(idx_hbm, ent_hbm, rel_hbm, time_hbm, pos_hbm, neg_hbm,
                  idx_v, h_v, t_v, nh_v, nt_v, r_v, ts_v, te_v,
                  psp_v, psn_v, pos_v, neg_v, sem):
    wid = lax.axis_index("s") * 2 + lax.axis_index("c")
    pltpu.sync_copy(idx_hbm.at[wid], idx_v)

    for c in range(NCHUNK):
        cps = [
            pltpu.async_copy(ent_hbm.at[idx_v.at[0, c]], h_v, sem),
            pltpu.async_copy(ent_hbm.at[idx_v.at[1, c]], t_v, sem),
            pltpu.async_copy(ent_hbm.at[idx_v.at[2, c]], nh_v, sem),
            pltpu.async_copy(ent_hbm.at[idx_v.at[3, c]], nt_v, sem),
            pltpu.async_copy(rel_hbm.at[idx_v.at[4, c]], r_v, sem),
            pltpu.async_copy(time_hbm.at[idx_v.at[5, c]], ts_v, sem),
            pltpu.async_copy(time_hbm.at[idx_v.at[6, c]], te_v, sem),
        ]
        for cp in cps:
            cp.wait()

        def body(i, _):
            accp = jnp.zeros((L,), jnp.float32)
            accn = jnp.zeros((L,), jnp.float32)
            for j in range(DJ):
                sl = pl.ds(j * L, L)
                trans = r_v[i, sl] + 0.5 * (ts_v[i, sl] + te_v[i, sl])
                accp = accp + jnp.abs(h_v[i, sl] + trans - t_v[i, sl])
                accn = accn + jnp.abs(nh_v[i, sl] + trans - nt_v[i, sl])
            psp_v[pl.ds(i * L, L)] = accp
            psn_v[pl.ds(i * L, L)] = accn
            return 0

        lax.fori_loop(0, C, body, 0)

        def body2(g, _, c=c):
            base = g * (L * L) + lax.iota(jnp.int32, L) * L
            sp = jnp.zeros((L,), jnp.float32)
            sn = jnp.zeros((L,), jnp.float32)
            for l in range(L):
                sp = sp + plsc.load_gather(psp_v, [base + l])
                sn = sn + plsc.load_gather(psn_v, [base + l])
            pos_v[pl.ds(c * C + g * L, L)] = -sp
            neg_v[pl.ds(c * C + g * L, L)] = -sn
            return 0

        lax.fori_loop(0, C // L, body2, 0)

    pltpu.sync_copy(pos_v, pos_hbm.at[pl.ds(wid * BPW, BPW)])
    pltpu.sync_copy(neg_v, neg_hbm.at[pl.ds(wid * BPW, BPW)])


def kernel(heads, tails, relations, start_time, end_time,
           negative_heads, negative_tails, ent_emb, rel_emb, time_emb):
    idx = jnp.stack([
        heads, tails, negative_heads, negative_tails,
        relations, start_time, end_time,
    ]).astype(jnp.int32).reshape(7, NW, NCHUNK, C).transpose(1, 0, 2, 3)
    return _score_kernel(idx, ent_emb, rel_emb, time_emb)


# speedup vs baseline: 1.2555x; 1.2555x over previous
import functools

import jax
import jax.numpy as jnp
from jax import lax
from jax.experimental import pallas as pl
from jax.experimental.pallas import tpu as pltpu
from jax.experimental.pallas import tpu_sc as plsc

B = 16384
D = 64
NW = 32
BPW = B // NW
C = 32
NCHUNK = BPW // C
L = 16
NG = C // L
N_REL = 500
N_TIME = 366

_mesh = plsc.VectorSubcoreMesh(core_axis_name="c", subcore_axis_name="s")


@functools.partial(
    pl.kernel,
    mesh=_mesh,
    compiler_params=pltpu.CompilerParams(needs_layout_passes=False),
    out_type=(
        jax.ShapeDtypeStruct((B,), jnp.float32),
        jax.ShapeDtypeStruct((B,), jnp.float32),
    ),
    scratch_types=[
        pltpu.VMEM((BPW,), jnp.int32),
        pltpu.VMEM((BPW,), jnp.int32),
        pltpu.VMEM((BPW,), jnp.int32),
        pltpu.VMEM((BPW,), jnp.int32),
        pltpu.VMEM((BPW,), jnp.int32),
        pltpu.VMEM((BPW,), jnp.int32),
        pltpu.VMEM((BPW,), jnp.int32),
        pltpu.VMEM((C, D), jnp.float32),
        pltpu.VMEM((C, D), jnp.float32),
        pltpu.VMEM((C, D), jnp.float32),
        pltpu.VMEM((C, D), jnp.float32),
        pltpu.VMEM((N_REL * D,), jnp.float32),
        pltpu.VMEM((N_TIME * D,), jnp.float32),
        pltpu.VMEM((BPW,), jnp.float32),
        pltpu.VMEM((BPW,), jnp.float32),
        pltpu.SemaphoreType.DMA,
    ],
)
def _score_kernel(h_hbm, t_hbm, nh_hbm, nt_hbm, r_hbm, st_hbm, et_hbm,
                  ent_hbm, rel_hbm, time_hbm, pos_hbm, neg_hbm,
                  hi_v, ti_v, nhi_v, nti_v, ri_v, si_v, ei_v,
                  h_v, t_v, nh_v, nt_v, rel_c, time_c,
                  pos_v, neg_v, sem):
    wid = lax.axis_index("s") * 2 + lax.axis_index("c")
    wb = wid * BPW
    pltpu.sync_copy(h_hbm.at[pl.ds(wb, BPW)], hi_v)
    pltpu.sync_copy(t_hbm.at[pl.ds(wb, BPW)], ti_v)
    pltpu.sync_copy(nh_hbm.at[pl.ds(wb, BPW)], nhi_v)
    pltpu.sync_copy(nt_hbm.at[pl.ds(wb, BPW)], nti_v)
    pltpu.sync_copy(r_hbm.at[pl.ds(wb, BPW)], ri_v)
    pltpu.sync_copy(st_hbm.at[pl.ds(wb, BPW)], si_v)
    pltpu.sync_copy(et_hbm.at[pl.ds(wb, BPW)], ei_v)
    pltpu.sync_copy(rel_hbm, rel_c)
    pltpu.sync_copy(time_hbm, time_c)

    def chunk_body(c, _):
        base = c * C
        for k in range(NG):
            hvec = hi_v[pl.ds(base + k * L, L)]
            tvec = ti_v[pl.ds(base + k * L, L)]
            nhvec = nhi_v[pl.ds(base + k * L, L)]
            ntvec = nti_v[pl.ds(base + k * L, L)]
            cps = []
            for l in range(L):
                slot = k * L + l
                cps.append(pltpu.async_copy(
                    ent_hbm.at[hvec[l]], h_v.at[slot], sem))
                cps.append(pltpu.async_copy(
                    ent_hbm.at[tvec[l]], t_v.at[slot], sem))
                cps.append(pltpu.async_copy(
                    ent_hbm.at[nhvec[l]], nh_v.at[slot], sem))
                cps.append(pltpu.async_copy(
                    ent_hbm.at[ntvec[l]], nt_v.at[slot], sem))
            for cp in cps:
                cp.wait()

        @plsc.parallel_loop(0, NG)
        def grp(g):
            sl = pl.ds(base + g * L, L)
            slot = lax.iota(jnp.int32, L) + g * L
            rbase = ri_v[sl] * D
            sbase = si_v[sl] * D
            ebase2 = ei_v[sl] * D
            accp = jnp.zeros((L,), jnp.float32)
            accn = jnp.zeros((L,), jnp.float32)
            for d in range(D):
                dv = jnp.full((L,), d, jnp.int32)
                hv = plsc.load_gather(h_v, [slot, dv])
                tv = plsc.load_gather(t_v, [slot, dv])
                nhv = plsc.load_gather(nh_v, [slot, dv])
                ntv = plsc.load_gather(nt_v, [slot, dv])
                rv = plsc.load_gather(rel_c, [rbase + d])
                tsv = plsc.load_gather(time_c, [sbase + d])
                tev = plsc.load_gather(time_c, [ebase2 + d])
                trans = rv + 0.5 * (tsv + tev)
                accp = accp + jnp.abs(hv + trans - tv)
                accn = accn + jnp.abs(nhv + trans - ntv)
            pos_v[sl] = -accp
            neg_v[sl] = -accn

        return 0

    lax.fori_loop(0, NCHUNK, chunk_body, 0)

    pltpu.sync_copy(pos_v, pos_hbm.at[pl.ds(wb, BPW)])
    pltpu.sync_copy(neg_v, neg_hbm.at[pl.ds(wb, BPW)])


def kernel(heads, tails, relations, start_time, end_time,
           negative_heads, negative_tails, ent_emb, rel_emb, time_emb):
    pos, neg = _score_kernel(
        heads.astype(jnp.int32), tails.astype(jnp.int32),
        negative_heads.astype(jnp.int32), negative_tails.astype(jnp.int32),
        relations.astype(jnp.int32), start_time.astype(jnp.int32),
        end_time.astype(jnp.int32),
        ent_emb, rel_emb.reshape(-1), time_emb.reshape(-1))
    return pos, neg
